# baseline (speedup 1.0000x reference)
import functools

import jax
import jax.numpy as jnp
from jax import lax
from jax.experimental import pallas as pl
from jax.experimental.pallas import tpu as pltpu
from jax.experimental.pallas import tpu_sc as plsc

CH = 128


def _dot_chunk(zb, wb, ob, o0):
    lane = lax.iota(jnp.int32, 16)
    last = lane == 15

    @plsc.parallel_loop(0, CH, 1, unroll=2)
    def body(i):
        acc = zb[i, pl.ds(0, 16)] * wb[i, pl.ds(0, 16)]
        for k in range(1, 8):
            acc = acc + zb[i, pl.ds(16 * k, 16)] * wb[i, pl.ds(16 * k, 16)]
        tot = plsc.cumsum(acc)
        plsc.store_scatter(ob, [jnp.full((16,), o0 + i, jnp.int32)], tot,
                           mask=last)


def kernel(Z, y, W1, b1):
    B, D = Z.shape
    info = plsc.get_sparse_core_info()
    nsub = info.num_subcores
    nw = info.num_cores * nsub
    bpw = B // nw
    nch = bpw // CH
    y32 = y.astype(jnp.int32)

    mesh = plsc.VectorSubcoreMesh(core_axis_name="c", subcore_axis_name="s")

    @functools.partial(
        pl.kernel,
        out_type=jax.ShapeDtypeStruct((B,), jnp.float32),
        mesh=mesh,
        compiler_params=pltpu.CompilerParams(needs_layout_passes=False),
        scratch_types=[
            pltpu.VMEM((bpw,), jnp.int32),
            pltpu.VMEM((2, CH, D), jnp.float32),
            pltpu.VMEM((2, CH, D), jnp.float32),
            pltpu.VMEM((bpw,), jnp.float32),
            pltpu.VMEM((bpw,), jnp.float32),
            pltpu.SemaphoreType.DMA((2,)),
            pltpu.SemaphoreType.DMA((2,)),
            pltpu.SemaphoreType.DMA,
            pltpu.SemaphoreType.DMA,
            pltpu.SemaphoreType.DMA,
            pltpu.SemaphoreType.DMA,
        ],
    )
    def k(z_hbm, y_hbm, w_hbm, b_hbm, out_hbm,
          idx_v, zbuf, wbuf, bbuf, obuf,
          zsem, wsem, bsem, ysem1, ysem2, osem):
        wid = lax.axis_index("c") * nsub + lax.axis_index("s")
        base0 = wid * bpw

        def zcopy(c, buf):
            return pltpu.make_async_copy(
                z_hbm.at[pl.ds(base0 + c * CH, CH)], zbuf.at[buf],
                zsem.at[buf])

        def wcopy(c, buf):
            return pltpu.make_async_copy(
                w_hbm.at[idx_v.at[pl.ds(c * CH, CH)]], wbuf.at[buf],
                wsem.at[buf])

        hy1 = pltpu.async_copy(y_hbm.at[pl.ds(base0, CH)],
                               idx_v.at[pl.ds(0, CH)], ysem1)
        hy2 = pltpu.async_copy(y_hbm.at[pl.ds(base0 + CH, bpw - CH)],
                               idx_v.at[pl.ds(CH, bpw - CH)], ysem2)
        zcopy(0, 0).start()
        zcopy(1, 1).start()
        hy1.wait()
        wcopy(0, 0).start()
        hy2.wait()
        for c in range(nch):
            pltpu.make_async_copy(b_hbm.at[idx_v.at[pl.ds(c * CH, CH)]],
                                  bbuf.at[pl.ds(c * CH, CH)], bsem).start()
        wcopy(1, 1).start()

        def chunk_body(c, carry):
            buf = c & 1
            zcopy(c, buf).wait()
            wcopy(c, buf).wait()
            o0 = c * CH
            _dot_chunk(zbuf.at[buf], wbuf.at[buf], obuf, o0)

            @pl.when(c + 2 < nch)
            def _():
                zcopy(c + 2, buf).start()
                wcopy(c + 2, buf).start()

            return carry

        lax.fori_loop(0, nch, chunk_body, 0)
        pltpu.make_async_copy(b_hbm.at[idx_v], bbuf, bsem).wait()
        for g in range(bpw // 16):
            obuf[pl.ds(g * 16, 16)] = (obuf[pl.ds(g * 16, 16)] +
                                       bbuf[pl.ds(g * 16, 16)])
        pltpu.sync_copy(obuf, out_hbm.at[pl.ds(base0, bpw)])

    return k(Z, y32, W1, b1)


# speedup vs baseline: 1.0058x; 1.0058x over previous
import functools

import jax
import jax.numpy as jnp
from jax import lax
from jax.experimental import pallas as pl
from jax.experimental.pallas import tpu as pltpu
from jax.experimental.pallas import tpu_sc as plsc

CH = 128


def _dot_chunk(zb, wb, ob, o0):
    lane = lax.iota(jnp.int32, 16)
    last = lane == 15

    @plsc.parallel_loop(0, CH, 1, unroll=1)
    def body(i):
        acc = zb[i, pl.ds(0, 16)] * wb[i, pl.ds(0, 16)]
        for k in range(1, 8):
            acc = acc + zb[i, pl.ds(16 * k, 16)] * wb[i, pl.ds(16 * k, 16)]
        tot = plsc.cumsum(acc)
        plsc.store_scatter(ob, [jnp.full((16,), o0 + i, jnp.int32)], tot,
                           mask=last)


def kernel(Z, y, W1, b1):
    B, D = Z.shape
    info = plsc.get_sparse_core_info()
    nsub = info.num_subcores
    nw = info.num_cores * nsub
    bpw = B // nw
    nch = bpw // CH
    y32 = y.astype(jnp.int32)

    mesh = plsc.VectorSubcoreMesh(core_axis_name="c", subcore_axis_name="s")

    @functools.partial(
        pl.kernel,
        out_type=jax.ShapeDtypeStruct((B,), jnp.float32),
        mesh=mesh,
        compiler_params=pltpu.CompilerParams(needs_layout_passes=False),
        scratch_types=[
            pltpu.VMEM((bpw,), jnp.int32),
            pltpu.VMEM((2, CH, D), jnp.float32),
            pltpu.VMEM((2, CH, D), jnp.float32),
            pltpu.VMEM((bpw,), jnp.float32),
            pltpu.VMEM((bpw,), jnp.float32),
            pltpu.SemaphoreType.DMA((2,)),
            pltpu.SemaphoreType.DMA((2,)),
            pltpu.SemaphoreType.DMA,
            pltpu.SemaphoreType.DMA,
            pltpu.SemaphoreType.DMA,
            pltpu.SemaphoreType.DMA,
        ],
    )
    def k(z_hbm, y_hbm, w_hbm, b_hbm, out_hbm,
          idx_v, zbuf, wbuf, bbuf, obuf,
          zsem, wsem, bsem, ysem1, ysem2, osem):
        wid = lax.axis_index("c") * nsub + lax.axis_index("s")
        base0 = wid * bpw

        def zcopy(c, buf):
            return pltpu.make_async_copy(
                z_hbm.at[pl.ds(base0 + c * CH, CH)], zbuf.at[buf],
                zsem.at[buf])

        def wcopy(c, buf):
            return pltpu.make_async_copy(
                w_hbm.at[idx_v.at[pl.ds(c * CH, CH)]], wbuf.at[buf],
                wsem.at[buf])

        hy1 = pltpu.async_copy(y_hbm.at[pl.ds(base0, CH)],
                               idx_v.at[pl.ds(0, CH)], ysem1)
        hy2 = pltpu.async_copy(y_hbm.at[pl.ds(base0 + CH, bpw - CH)],
                               idx_v.at[pl.ds(CH, bpw - CH)], ysem2)
        zcopy(0, 0).start()
        zcopy(1, 1).start()
        hy1.wait()
        wcopy(0, 0).start()
        hy2.wait()
        for c in range(nch):
            pltpu.make_async_copy(b_hbm.at[idx_v.at[pl.ds(c * CH, CH)]],
                                  bbuf.at[pl.ds(c * CH, CH)], bsem).start()
        wcopy(1, 1).start()

        def chunk_body(c, carry):
            buf = c & 1
            zcopy(c, buf).wait()
            wcopy(c, buf).wait()
            o0 = c * CH
            _dot_chunk(zbuf.at[buf], wbuf.at[buf], obuf, o0)

            @pl.when(c + 2 < nch)
            def _():
                zcopy(c + 2, buf).start()
                wcopy(c + 2, buf).start()

            return carry

        lax.fori_loop(0, nch, chunk_body, 0)
        pltpu.make_async_copy(b_hbm.at[idx_v], bbuf, bsem).wait()
        for g in range(bpw // 16):
            obuf[pl.ds(g * 16, 16)] = (obuf[pl.ds(g * 16, 16)] +
                                       bbuf[pl.ds(g * 16, 16)])
        pltpu.sync_copy(obuf, out_hbm.at[pl.ds(base0, bpw)])

    return k(Z, y32, W1, b1)
